# baseline (speedup 1.0000x reference)
import jax
import jax.numpy as jnp
from jax import lax
from jax.experimental import pallas as pl
from jax.experimental.pallas import tpu as pltpu
from jax.experimental.pallas import tpu_sc as plsc

N = 10000
E = 320000
G = 64
LATENT = 64
EDGE_F = 16

NC = 2
NS = 16
NW = NC * NS
B = 80
ROWS = E // B
RPW = ROWS // NW
NPAD = 10240
RPT = NPAD // NS
NBUF = 5
NGRP = RPW // NBUF

EPB = 80
EPROWS = E // EPB
EPW = EPROWS // NW
EFR = EPB * EDGE_F // 128
NBUF_E = 5
EPG = EPW // NBUF_E


def _sc_round_body(cur_hbm, ei_hbm, zeros_hbm, out_hbm,
                   acc_sh, cur_sh, src_v, dst_v, rows_v, gsem, ssem):
    c = lax.axis_index("c")
    s = lax.axis_index("s")
    wid = s * NC + c
    base = wid * RPW
    pltpu.sync_copy(ei_hbm.at[0].at[pl.ds(base, RPW)], src_v)
    pltpu.sync_copy(ei_hbm.at[1].at[pl.ds(base, RPW)], dst_v)
    pltpu.sync_copy(cur_hbm.at[pl.ds(s * (N // NS), N // NS)],
                    cur_sh.at[pl.ds(s * (N // NS), N // NS)])
    pltpu.sync_copy(zeros_hbm.at[pl.ds(s * RPT, RPT)],
                    acc_sh.at[pl.ds(s * RPT, RPT)])
    plsc.subcore_barrier()

    def group(i, _):
        @pl.when(i > 0)
        def _():
            for b in range(NBUF):
                pltpu.make_async_copy(
                    rows_v.at[b], acc_sh.at[dst_v.at[i * NBUF + b - NBUF]],
                    ssem.at[b]).wait()
        for b in range(NBUF):
            pltpu.async_copy(cur_sh.at[src_v.at[i * NBUF + b]],
                             rows_v.at[b], gsem.at[b])
        for b in range(NBUF):
            j = i * NBUF + b
            pltpu.make_async_copy(cur_sh.at[src_v.at[j]],
                                  rows_v.at[b], gsem.at[b]).wait()
            pltpu.async_copy(rows_v.at[b], acc_sh.at[dst_v.at[j]],
                             ssem.at[b], add=True)
        return 0

    lax.fori_loop(0, NGRP, group, 0)
    for b in range(NBUF):
        pltpu.make_async_copy(
            rows_v.at[b], acc_sh.at[dst_v.at[(NGRP - 1) * NBUF + b]],
            ssem.at[b]).wait()
    plsc.subcore_barrier()
    pltpu.sync_copy(acc_sh.at[pl.ds(s * RPT, RPT)],
                    out_hbm.at[c].at[pl.ds(s * RPT, RPT)])


def _sc_epool_body(ef_hbm, ei_hbm, zeros_hbm, out_hbm,
                   acc_sh, efa_v, efb_v, dst_v, gsem, ssem):
    c = lax.axis_index("c")
    s = lax.axis_index("s")
    wid = s * NC + c
    base = wid * EPW
    pltpu.sync_copy(ei_hbm.at[1].at[pl.ds(base, EPW)], dst_v)
    pltpu.sync_copy(zeros_hbm.at[pl.ds(s * RPT, RPT)],
                    acc_sh.at[pl.ds(s * RPT, RPT)])
    plsc.subcore_barrier()

    def repack(b):
        for e in range(EPB):
            efb_v[b, e, :] = efa_v[b, e // 8, pl.ds(16 * (e % 8), 16)]

    def group(i, _):
        @pl.when(i > 0)
        def _():
            for b in range(NBUF_E):
                pltpu.make_async_copy(
                    efb_v.at[b], acc_sh.at[dst_v.at[i * NBUF_E + b - NBUF_E]],
                    ssem.at[b]).wait()
        for b in range(NBUF_E):
            pltpu.async_copy(ef_hbm.at[pl.ds((base + i * NBUF_E + b) * EFR,
                                             EFR)],
                             efa_v.at[b], gsem.at[b])
        for b in range(NBUF_E):
            j = i * NBUF_E + b
            pltpu.make_async_copy(ef_hbm.at[pl.ds((base + j) * EFR, EFR)],
                                  efa_v.at[b], gsem.at[b]).wait()
            repack(b)
            pltpu.async_copy(efb_v.at[b], acc_sh.at[dst_v.at[j]],
                             ssem.at[b], add=True)
        return 0

    lax.fori_loop(0, EPG, group, 0)
    for b in range(NBUF_E):
        pltpu.make_async_copy(
            efb_v.at[b], acc_sh.at[dst_v.at[(EPG - 1) * NBUF_E + b]],
            ssem.at[b]).wait()
    plsc.subcore_barrier()
    pltpu.sync_copy(acc_sh.at[pl.ds(s * RPT, RPT)],
                    out_hbm.at[c].at[pl.ds(s * RPT, RPT)])


_SC_MESH = plsc.VectorSubcoreMesh(core_axis_name="c", subcore_axis_name="s")
_SC_PARAMS = pltpu.CompilerParams(use_tc_tiling_on_sc=False)

_sc_round = pl.kernel(
    _sc_round_body,
    out_type=jax.ShapeDtypeStruct((NC, NPAD, LATENT), jnp.float32),
    mesh=_SC_MESH,
    scratch_types=[
        pltpu.VMEM_SHARED((NPAD, LATENT), jnp.float32),
        pltpu.VMEM_SHARED((N, LATENT), jnp.float32),
        pltpu.VMEM((RPW, B), jnp.int32),
        pltpu.VMEM((RPW, B), jnp.int32),
        pltpu.VMEM((NBUF, B, LATENT), jnp.float32),
        pltpu.SemaphoreType.DMA((NBUF,)),
        pltpu.SemaphoreType.DMA((NBUF,)),
    ],
    compiler_params=_SC_PARAMS,
)

_sc_epool = pl.kernel(
    _sc_epool_body,
    out_type=jax.ShapeDtypeStruct((NC, NPAD, EDGE_F), jnp.float32),
    mesh=_SC_MESH,
    scratch_types=[
        pltpu.VMEM_SHARED((NPAD, EDGE_F), jnp.float32),
        pltpu.VMEM((NBUF_E, EFR, 128), jnp.float32),
        pltpu.VMEM((NBUF_E, EPB, EDGE_F), jnp.float32),
        pltpu.VMEM((EPW, EPB), jnp.int32),
        pltpu.SemaphoreType.DMA((NBUF_E,)),
        pltpu.SemaphoreType.DMA((NBUF_E,)),
    ],
    compiler_params=_SC_PARAMS,
)


def _tc_prep_body(nf, w_n2l, seg, w_e2l, im_ref, cur_ref):
    e2n = jnp.dot(seg[0, :N] + seg[1, :N], w_e2l[...],
                  preferred_element_type=jnp.float32)
    im = jnp.dot(nf[...], w_n2l[...],
                 preferred_element_type=jnp.float32) + e2n
    im_ref[...] = im
    cur_ref[...] = jnp.maximum(im, 0.0)


def _tc_round_body(part, conv, im, cur_ref):
    n2n = part[0, :N] + part[1, :N]
    lin = jnp.dot(n2n, conv[...], preferred_element_type=jnp.float32)
    cur_ref[...] = jnp.maximum(lin + im[...], 0.0)


def _tc_final_body(cur, out_params, gids, labels, w1, b1, w2, b2,
                   pred_ref, mse_ref):
    ne = jnp.maximum(
        jnp.dot(cur[...], out_params[...], preferred_element_type=jnp.float32),
        0.0)
    gi = lax.broadcasted_iota(jnp.int32, (G, N), 0)
    onehot = (gi == gids[...]).astype(jnp.float32)
    ge = jnp.dot(onehot, ne, preferred_element_type=jnp.float32)
    h = jnp.maximum(
        jnp.dot(ge, w1[...], preferred_element_type=jnp.float32) + b1[...],
        0.0)
    pred = jnp.dot(h, w2[...], preferred_element_type=jnp.float32) + b2[...]
    pred_ref[...] = pred
    diff = pred - labels[...]
    mse_ref[...] = jnp.mean(diff * diff).reshape(1, 1)


_tc_prep = pl.pallas_call(
    _tc_prep_body,
    out_shape=(jax.ShapeDtypeStruct((N, LATENT), jnp.float32),
               jax.ShapeDtypeStruct((N, LATENT), jnp.float32)),
)

_tc_round = pl.pallas_call(
    _tc_round_body,
    out_shape=jax.ShapeDtypeStruct((N, LATENT), jnp.float32),
)

_tc_final = pl.pallas_call(
    _tc_final_body,
    out_shape=(jax.ShapeDtypeStruct((G, 1), jnp.float32),
               jax.ShapeDtypeStruct((1, 1), jnp.float32)),
)


def kernel(node_feat, edge_feat, edge_index, graph_ids, labels,
           w_n2l, w_e2l, conv_params, out_params, w1, b1, w2, b2):
    ei50 = edge_index.reshape(2, ROWS, B)
    ei80 = edge_index.reshape(2, EPROWS, EPB)
    ef128 = edge_feat.reshape(E * EDGE_F // 128, 128)
    zeros64 = jnp.zeros((NPAD, LATENT), jnp.float32)
    zeros16 = jnp.zeros((NPAD, EDGE_F), jnp.float32)

    seg = _sc_epool(ef128, ei80, zeros16)
    im, cur = _tc_prep(node_feat, w_n2l, seg, w_e2l)
    for _ in range(3):
        part = _sc_round(cur, ei50, zeros64)
        cur = _tc_round(part, conv_params, im)
    pred, mse = _tc_final(cur, out_params, graph_ids.reshape(1, N),
                          labels, w1, b1.reshape(1, -1), w2,
                          b2.reshape(1, 1))
    return pred, mse.reshape(())


# speedup vs baseline: 1.0535x; 1.0535x over previous
import jax
import jax.numpy as jnp
from jax import lax
from jax.experimental import pallas as pl
from jax.experimental.pallas import tpu as pltpu
from jax.experimental.pallas import tpu_sc as plsc

N = 10000
E = 320000
G = 64
LATENT = 64
EDGE_F = 16

NC = 2
NS = 16
NW = NC * NS
B = 100
ROWS = E // B
RPW = ROWS // NW
NPAD = 10240
RPT = NPAD // NS
NBUF = 4
NGRP = RPW // NBUF

EPB = 80
EPROWS = E // EPB
EPW = EPROWS // NW
EFR = EPB * EDGE_F // 128
NBUF_E = 5
EPG = EPW // NBUF_E


def _sc_round_body(cur_hbm, ei_hbm, zeros_hbm, out_hbm,
                   acc_sh, cur_sh, src_v, dst_v, rows_v, gsem, ssem):
    c = lax.axis_index("c")
    s = lax.axis_index("s")
    wid = s * NC + c
    base = wid * RPW
    pltpu.sync_copy(ei_hbm.at[0].at[pl.ds(base, RPW)], src_v)
    pltpu.sync_copy(ei_hbm.at[1].at[pl.ds(base, RPW)], dst_v)
    pltpu.sync_copy(cur_hbm.at[pl.ds(s * (N // NS), N // NS)],
                    cur_sh.at[pl.ds(s * (N // NS), N // NS)])
    pltpu.sync_copy(zeros_hbm.at[pl.ds(s * RPT, RPT)],
                    acc_sh.at[pl.ds(s * RPT, RPT)])
    plsc.subcore_barrier()

    def group(i, _):
        @pl.when(i > 0)
        def _():
            for b in range(NBUF):
                pltpu.make_async_copy(
                    rows_v.at[b], acc_sh.at[dst_v.at[i * NBUF + b - NBUF]],
                    ssem.at[b]).wait()
        for b in range(NBUF):
            pltpu.async_copy(cur_sh.at[src_v.at[i * NBUF + b]],
                             rows_v.at[b], gsem.at[b])
        for b in range(NBUF):
            j = i * NBUF + b
            pltpu.make_async_copy(cur_sh.at[src_v.at[j]],
                                  rows_v.at[b], gsem.at[b]).wait()
            pltpu.async_copy(rows_v.at[b], acc_sh.at[dst_v.at[j]],
                             ssem.at[b], add=True)
        return 0

    lax.fori_loop(0, NGRP, group, 0)
    for b in range(NBUF):
        pltpu.make_async_copy(
            rows_v.at[b], acc_sh.at[dst_v.at[(NGRP - 1) * NBUF + b]],
            ssem.at[b]).wait()
    plsc.subcore_barrier()
    pltpu.sync_copy(acc_sh.at[pl.ds(s * RPT, RPT)],
                    out_hbm.at[c].at[pl.ds(s * RPT, RPT)])


def _sc_epool_body(ef_hbm, ei_hbm, zeros_hbm, out_hbm,
                   acc_sh, efa_v, efb_v, dst_v, gsem, ssem):
    c = lax.axis_index("c")
    s = lax.axis_index("s")
    wid = s * NC + c
    base = wid * EPW
    pltpu.sync_copy(ei_hbm.at[1].at[pl.ds(base, EPW)], dst_v)
    pltpu.sync_copy(zeros_hbm.at[pl.ds(s * RPT, RPT)],
                    acc_sh.at[pl.ds(s * RPT, RPT)])
    plsc.subcore_barrier()

    def repack(b):
        for e in range(EPB):
            efb_v[b, e, :] = efa_v[b, e // 8, pl.ds(16 * (e % 8), 16)]

    def group(i, _):
        @pl.when(i > 0)
        def _():
            for b in range(NBUF_E):
                pltpu.make_async_copy(
                    efb_v.at[b], acc_sh.at[dst_v.at[i * NBUF_E + b - NBUF_E]],
                    ssem.at[b]).wait()
        for b in range(NBUF_E):
            pltpu.async_copy(ef_hbm.at[pl.ds((base + i * NBUF_E + b) * EFR,
                                             EFR)],
                             efa_v.at[b], gsem.at[b])
        for b in range(NBUF_E):
            j = i * NBUF_E + b
            pltpu.make_async_copy(ef_hbm.at[pl.ds((base + j) * EFR, EFR)],
                                  efa_v.at[b], gsem.at[b]).wait()
            repack(b)
            pltpu.async_copy(efb_v.at[b], acc_sh.at[dst_v.at[j]],
                             ssem.at[b], add=True)
        return 0

    lax.fori_loop(0, EPG, group, 0)
    for b in range(NBUF_E):
        pltpu.make_async_copy(
            efb_v.at[b], acc_sh.at[dst_v.at[(EPG - 1) * NBUF_E + b]],
            ssem.at[b]).wait()
    plsc.subcore_barrier()
    pltpu.sync_copy(acc_sh.at[pl.ds(s * RPT, RPT)],
                    out_hbm.at[c].at[pl.ds(s * RPT, RPT)])


_SC_MESH = plsc.VectorSubcoreMesh(core_axis_name="c", subcore_axis_name="s")
_SC_PARAMS = pltpu.CompilerParams(use_tc_tiling_on_sc=False)

_sc_round = pl.kernel(
    _sc_round_body,
    out_type=jax.ShapeDtypeStruct((NC, NPAD, LATENT), jnp.float32),
    mesh=_SC_MESH,
    scratch_types=[
        pltpu.VMEM_SHARED((NPAD, LATENT), jnp.float32),
        pltpu.VMEM_SHARED((N, LATENT), jnp.float32),
        pltpu.VMEM((RPW, B), jnp.int32),
        pltpu.VMEM((RPW, B), jnp.int32),
        pltpu.VMEM((NBUF, B, LATENT), jnp.float32),
        pltpu.SemaphoreType.DMA((NBUF,)),
        pltpu.SemaphoreType.DMA((NBUF,)),
    ],
    compiler_params=_SC_PARAMS,
)

_sc_epool = pl.kernel(
    _sc_epool_body,
    out_type=jax.ShapeDtypeStruct((NC, NPAD, EDGE_F), jnp.float32),
    mesh=_SC_MESH,
    scratch_types=[
        pltpu.VMEM_SHARED((NPAD, EDGE_F), jnp.float32),
        pltpu.VMEM((NBUF_E, EFR, 128), jnp.float32),
        pltpu.VMEM((NBUF_E, EPB, EDGE_F), jnp.float32),
        pltpu.VMEM((EPW, EPB), jnp.int32),
        pltpu.SemaphoreType.DMA((NBUF_E,)),
        pltpu.SemaphoreType.DMA((NBUF_E,)),
    ],
    compiler_params=_SC_PARAMS,
)


def _tc_prep_body(nf, w_n2l, seg, w_e2l, im_ref, cur_ref):
    e2n = jnp.dot(seg[0, :N] + seg[1, :N], w_e2l[...],
                  preferred_element_type=jnp.float32)
    im = jnp.dot(nf[...], w_n2l[...],
                 preferred_element_type=jnp.float32) + e2n
    im_ref[...] = im
    cur_ref[...] = jnp.maximum(im, 0.0)


def _tc_round_body(part, conv, im, cur_ref):
    n2n = part[0, :N] + part[1, :N]
    lin = jnp.dot(n2n, conv[...], preferred_element_type=jnp.float32)
    cur_ref[...] = jnp.maximum(lin + im[...], 0.0)


def _tc_final_body(cur, out_params, gids, labels, w1, b1, w2, b2,
                   pred_ref, mse_ref):
    ne = jnp.maximum(
        jnp.dot(cur[...], out_params[...], preferred_element_type=jnp.float32),
        0.0)
    gi = lax.broadcasted_iota(jnp.int32, (G, N), 0)
    onehot = (gi == gids[...]).astype(jnp.float32)
    ge = jnp.dot(onehot, ne, preferred_element_type=jnp.float32)
    h = jnp.maximum(
        jnp.dot(ge, w1[...], preferred_element_type=jnp.float32) + b1[...],
        0.0)
    pred = jnp.dot(h, w2[...], preferred_element_type=jnp.float32) + b2[...]
    pred_ref[...] = pred
    diff = pred - labels[...]
    mse_ref[...] = jnp.mean(diff * diff).reshape(1, 1)


_tc_prep = pl.pallas_call(
    _tc_prep_body,
    out_shape=(jax.ShapeDtypeStruct((N, LATENT), jnp.float32),
               jax.ShapeDtypeStruct((N, LATENT), jnp.float32)),
)

_tc_round = pl.pallas_call(
    _tc_round_body,
    out_shape=jax.ShapeDtypeStruct((N, LATENT), jnp.float32),
)

_tc_final = pl.pallas_call(
    _tc_final_body,
    out_shape=(jax.ShapeDtypeStruct((G, 1), jnp.float32),
               jax.ShapeDtypeStruct((1, 1), jnp.float32)),
)


def kernel(node_feat, edge_feat, edge_index, graph_ids, labels,
           w_n2l, w_e2l, conv_params, out_params, w1, b1, w2, b2):
    ei50 = edge_index.reshape(2, ROWS, B)
    ei80 = edge_index.reshape(2, EPROWS, EPB)
    ef128 = edge_feat.reshape(E * EDGE_F // 128, 128)
    zeros64 = jnp.zeros((NPAD, LATENT), jnp.float32)
    zeros16 = jnp.zeros((NPAD, EDGE_F), jnp.float32)

    seg = _sc_epool(ef128, ei80, zeros16)
    im, cur = _tc_prep(node_feat, w_n2l, seg, w_e2l)
    for _ in range(3):
        part = _sc_round(cur, ei50, zeros64)
        cur = _tc_round(part, conv_params, im)
    pred, mse = _tc_final(cur, out_params, graph_ids.reshape(1, N),
                          labels, w1, b1.reshape(1, -1), w2,
                          b2.reshape(1, 1))
    return pred, mse.reshape(())
